# Initial kernel scaffold; baseline (speedup 1.0000x reference)
import jax
import jax.numpy as jnp
from jax.experimental import pallas as pl


def kernel(preg, pcls, ancs_xywh, gboxes_ltrb, glabels):
    raise NotImplementedError("write your pallas kernel here")


# speedup vs baseline: 15.2946x; 15.2946x over previous
import jax
import jax.numpy as jnp
from jax.experimental import pallas as pl
from jax.experimental.pallas import tpu as pltpu

_NUM_CLASSES = 20
_IOU_T = 0.5
_NEG_RATIO = 3.0
_VXY = 0.1
_VWH = 0.2
_EPS16 = 9.765625e-4


def _ssd_body(anc_ref, preg_ref, pcls_ref, gt_ref, out_ref):
    A = anc_ref.shape[1]
    G = gt_ref.shape[1]
    C = pcls_ref.shape[1]

    anc = anc_ref[...]
    acx, acy, aw, ah = anc[0:1], anc[1:2], anc[2:3], anc[3:4]
    al = acx - aw * 0.5
    at = acy - ah * 0.5
    ar = acx + aw * 0.5
    ab = acy + ah * 0.5
    area_a = (ar - al) * (ab - at)

    gt = gt_ref[0]
    gl = gt[:, 0:1]
    gtp = gt[:, 1:2]
    gr = gt[:, 2:3]
    gb = gt[:, 3:4]
    glab = gt[:, 4:5]

    w = jnp.maximum(jnp.minimum(gr, ar) - jnp.maximum(gl, al), 0.0)
    h = jnp.maximum(jnp.minimum(gb, ab) - jnp.maximum(gtp, at), 0.0)
    inter = w * h
    area_g = (gr - gl) * (gb - gtp)
    iou = inter / jnp.maximum(area_g + area_a - inter, 1e-8)
    iou = jnp.where(glab > 0.0, iou, -1.0)

    best = jnp.max(iou, axis=0, keepdims=True)
    mask_pos = best >= _IOU_T

    gidx = jax.lax.broadcasted_iota(jnp.int32, (G, A), 0)
    best_idx = jnp.min(jnp.where(iou == best, gidx, G), axis=0,
                       keepdims=True)
    sel = gidx == best_idx

    def take(col):
        return jnp.sum(jnp.where(sel, col, 0.0), axis=0, keepdims=True)

    gcx = take((gl + gr) * 0.5)
    gcy = take((gtp + gb) * 0.5)
    gw = take(gr - gl)
    gh = take(gb - gtp)
    glabel = jnp.where(mask_pos, take(glab), 0.0)

    tx = (gcx - acx) / (aw * _VXY)
    ty = (gcy - acy) / (ah * _VXY)
    tw = jnp.log(jnp.maximum(gw, 1e-6) / aw) / _VWH
    th = jnp.log(jnp.maximum(gh, 1e-6) / ah) / _VWH
    d = preg_ref[0] - jnp.concatenate([tx, ty, tw, th], axis=0)
    ad = jnp.abs(d)
    sl1 = jnp.sum(jnp.where(ad < 1.0, 0.5 * d * d, ad - 0.5), axis=0,
                  keepdims=True)

    n_pos = jnp.sum(jnp.where(mask_pos, 1.0, 0.0))
    sl1_pos = jnp.sum(jnp.where(mask_pos, sl1, 0.0))

    pc = pcls_ref[0]
    mx = jnp.max(pc, axis=0, keepdims=True)
    lse = mx + jnp.log(jnp.sum(jnp.exp(pc - mx), axis=0, keepdims=True))
    cidx = jax.lax.broadcasted_iota(jnp.int32, (C, A), 0)
    p_at = jnp.sum(jnp.where(cidx == glabel.astype(jnp.int32), pc, 0.0),
                   axis=0, keepdims=True)
    ce = lse - p_at

    ce_pos = jnp.sum(jnp.where(mask_pos, ce, 0.0))

    v = jnp.maximum(jnp.where(mask_pos, 0.0, ce), 0.0)
    vb = pltpu.bitcast(v, jnp.int32)

    kf = _NEG_RATIO * jnp.maximum(n_pos, _EPS16)
    kfl = jnp.floor(kf)
    k = kfl + jnp.where(kf > kfl, 1.0, 0.0)

    def bit_step(i, p):
        cand = p | jax.lax.shift_left(jnp.int32(1), jnp.int32(30) - i)
        cnt = jnp.sum(jnp.where(vb >= cand, 1.0, 0.0))
        return jax.lax.select(cnt >= k, cand, p)

    p_th = jax.lax.fori_loop(0, 31, bit_step, jnp.int32(0))

    cnt_gt = jnp.sum(jnp.where(vb > p_th, 1.0, 0.0))
    sum_gt = jnp.sum(jnp.where(vb > p_th, v, 0.0))
    t = jnp.max(jnp.where(vb <= p_th, v, 0.0))
    neg_sum = sum_gt + (k - cnt_gt) * t

    lane = jax.lax.broadcasted_iota(jnp.int32, (1, 8), 1)
    out_ref[0] = (jnp.where(lane == 0, n_pos, 0.0)
                  + jnp.where(lane == 1, sl1_pos, 0.0)
                  + jnp.where(lane == 2, ce_pos, 0.0)
                  + jnp.where(lane == 3, neg_sum, 0.0))


def kernel(preg, pcls, ancs_xywh, gboxes_ltrb, glabels):
    B, _, A = preg.shape
    C = pcls.shape[1]
    G = gboxes_ltrb.shape[1]
    anc_t = ancs_xywh.T
    gt = jnp.concatenate(
        [gboxes_ltrb, glabels[..., None].astype(jnp.float32)], axis=-1)

    out = pl.pallas_call(
        _ssd_body,
        grid=(B,),
        in_specs=[
            pl.BlockSpec((4, A), lambda b: (0, 0)),
            pl.BlockSpec((1, 4, A), lambda b: (b, 0, 0)),
            pl.BlockSpec((1, C, A), lambda b: (b, 0, 0)),
            pl.BlockSpec((1, G, 5), lambda b: (b, 0, 0)),
        ],
        out_specs=pl.BlockSpec((1, 1, 8), lambda b: (b, 0, 0)),
        out_shape=jax.ShapeDtypeStruct((B, 1, 8), jnp.float32),
        compiler_params=pltpu.CompilerParams(
            dimension_semantics=("parallel",),
            vmem_limit_bytes=56 * 1024 * 1024,
        ),
    )(anc_t, preg, pcls, gt)

    r = out[:, 0, :]
    n_pos = r[:, 0]
    l_box = r[:, 1].sum() / jnp.maximum(n_pos.sum(), 1.0)
    nums = jnp.maximum(n_pos, _EPS16)
    return l_box + (r[:, 2] / nums).mean() + (r[:, 3] / nums).mean()


# speedup vs baseline: 23.7534x; 1.5531x over previous
import jax
import jax.numpy as jnp
from jax.experimental import pallas as pl
from jax.experimental.pallas import tpu as pltpu

_NUM_CLASSES = 20
_IOU_T = 0.5
_NEG_RATIO = 3.0
_VXY = 0.1
_VWH = 0.2
_EPS16 = 9.765625e-4
_IMGS = 8


def _ssd_body(anc_ref, preg_ref, pcls_ref, gt_ref, out_ref):
    A = anc_ref.shape[1]
    G = gt_ref.shape[1]
    C = pcls_ref.shape[1]

    anc = anc_ref[...]
    acx, acy, aw, ah = anc[0:1], anc[1:2], anc[2:3], anc[3:4]
    al = acx - aw * 0.5
    at = acy - ah * 0.5
    ar = acx + aw * 0.5
    ab = acy + ah * 0.5
    area_a = (ar - al) * (ab - at)

    v_rows = []
    np_rows = []
    sl1_rows = []
    cep_rows = []
    for i in range(_IMGS):
        gt = gt_ref[i]
        gl = gt[:, 0:1]
        gtp = gt[:, 1:2]
        gr = gt[:, 2:3]
        gb = gt[:, 3:4]
        glab = gt[:, 4:5]

        w = jnp.maximum(jnp.minimum(gr, ar) - jnp.maximum(gl, al), 0.0)
        h = jnp.maximum(jnp.minimum(gb, ab) - jnp.maximum(gtp, at), 0.0)
        inter = w * h
        area_g = (gr - gl) * (gb - gtp)
        iou = inter / jnp.maximum(area_g + area_a - inter, 1e-8)
        iou = jnp.where(glab > 0.0, iou, -1.0)

        best = jnp.max(iou, axis=0, keepdims=True)
        mask_pos = best >= _IOU_T

        gidx = jax.lax.broadcasted_iota(jnp.int32, (G, A), 0)
        best_idx = jnp.min(jnp.where(iou == best, gidx, G), axis=0,
                           keepdims=True)
        sel = gidx == best_idx

        def take(col):
            return jnp.sum(jnp.where(sel, col, 0.0), axis=0, keepdims=True)

        gcx = take((gl + gr) * 0.5)
        gcy = take((gtp + gb) * 0.5)
        gw = take(gr - gl)
        gh = take(gb - gtp)
        glabel = jnp.where(mask_pos, take(glab), 0.0)

        tx = (gcx - acx) / (aw * _VXY)
        ty = (gcy - acy) / (ah * _VXY)
        tw = jnp.log(jnp.maximum(gw, 1e-6) / aw) / _VWH
        th = jnp.log(jnp.maximum(gh, 1e-6) / ah) / _VWH
        d = preg_ref[i] - jnp.concatenate([tx, ty, tw, th], axis=0)
        ad = jnp.abs(d)
        sl1 = jnp.sum(jnp.where(ad < 1.0, 0.5 * d * d, ad - 0.5), axis=0,
                      keepdims=True)

        pc = pcls_ref[i]
        mx = jnp.max(pc, axis=0, keepdims=True)
        lse = mx + jnp.log(jnp.sum(jnp.exp(pc - mx), axis=0, keepdims=True))
        cidx = jax.lax.broadcasted_iota(jnp.int32, (C, A), 0)
        p_at = jnp.sum(jnp.where(cidx == glabel.astype(jnp.int32), pc, 0.0),
                       axis=0, keepdims=True)
        ce = lse - p_at

        np_rows.append(jnp.sum(jnp.where(mask_pos, 1.0, 0.0), axis=1,
                               keepdims=True))
        sl1_rows.append(jnp.sum(jnp.where(mask_pos, sl1, 0.0), axis=1,
                                keepdims=True))
        cep_rows.append(jnp.sum(jnp.where(mask_pos, ce, 0.0), axis=1,
                                keepdims=True))
        v_rows.append(jnp.maximum(jnp.where(mask_pos, 0.0, ce), 0.0))

    v8 = jnp.concatenate(v_rows, axis=0)
    np8 = jnp.concatenate(np_rows, axis=0)
    sl18 = jnp.concatenate(sl1_rows, axis=0)
    cep8 = jnp.concatenate(cep_rows, axis=0)

    vb8 = pltpu.bitcast(v8, jnp.int32)
    kf = _NEG_RATIO * jnp.maximum(np8, _EPS16)
    kfl = jnp.floor(kf)
    k8 = kfl + jnp.where(kf > kfl, 1.0, 0.0)

    def bit_step(i, p):
        cand = p | jax.lax.shift_left(jnp.int32(1), jnp.int32(30) - i)
        cnt = jnp.sum(jnp.where(vb8 >= cand, 1.0, 0.0), axis=1, keepdims=True)
        return jnp.where(cnt >= k8, cand, p)

    p8 = jax.lax.fori_loop(0, 31, bit_step, jnp.zeros((_IMGS, 1), jnp.int32))

    cnt_gt = jnp.sum(jnp.where(vb8 > p8, 1.0, 0.0), axis=1, keepdims=True)
    sum_gt = jnp.sum(jnp.where(vb8 > p8, v8, 0.0), axis=1, keepdims=True)
    t8 = jnp.max(jnp.where(vb8 <= p8, v8, 0.0), axis=1, keepdims=True)
    neg8 = sum_gt + (k8 - cnt_gt) * t8

    lane = jax.lax.broadcasted_iota(jnp.int32, (_IMGS, 8), 1)
    out_ref[:, 0, :] = (jnp.where(lane == 0, np8, 0.0)
                        + jnp.where(lane == 1, sl18, 0.0)
                        + jnp.where(lane == 2, cep8, 0.0)
                        + jnp.where(lane == 3, neg8, 0.0))


def kernel(preg, pcls, ancs_xywh, gboxes_ltrb, glabels):
    B, _, A = preg.shape
    C = pcls.shape[1]
    G = gboxes_ltrb.shape[1]
    anc_t = ancs_xywh.T
    gt = jnp.concatenate(
        [gboxes_ltrb, glabels[..., None].astype(jnp.float32)], axis=-1)

    out = pl.pallas_call(
        _ssd_body,
        grid=(B // _IMGS,),
        in_specs=[
            pl.BlockSpec((4, A), lambda b: (0, 0)),
            pl.BlockSpec((_IMGS, 4, A), lambda b: (b, 0, 0)),
            pl.BlockSpec((_IMGS, C, A), lambda b: (b, 0, 0)),
            pl.BlockSpec((_IMGS, G, 5), lambda b: (b, 0, 0)),
        ],
        out_specs=pl.BlockSpec((_IMGS, 1, 8), lambda b: (b, 0, 0)),
        out_shape=jax.ShapeDtypeStruct((B, 1, 8), jnp.float32),
        compiler_params=pltpu.CompilerParams(
            dimension_semantics=("parallel",),
            vmem_limit_bytes=56 * 1024 * 1024,
        ),
    )(anc_t, preg, pcls, gt)

    r = out[:, 0, :]
    n_pos = r[:, 0]
    l_box = r[:, 1].sum() / jnp.maximum(n_pos.sum(), 1.0)
    nums = jnp.maximum(n_pos, _EPS16)
    return l_box + (r[:, 2] / nums).mean() + (r[:, 3] / nums).mean()


# speedup vs baseline: 43.6901x; 1.8393x over previous
import jax
import jax.numpy as jnp
from jax.experimental import pallas as pl
from jax.experimental.pallas import tpu as pltpu

_NUM_CLASSES = 20
_IOU_T = 0.5
_NEG_RATIO = 3.0
_VXY = 0.1
_VWH = 0.2
_EPS16 = 9.765625e-4
_IMGS = 8


def _ssd_body(anc_ref, preg_ref, pcls_ref, gt_ref, gtt_ref, out_ref):
    A = anc_ref.shape[1]
    G = gt_ref.shape[1]
    C = pcls_ref.shape[1]

    anc = anc_ref[...]
    acx, acy, aw, ah = anc[0:1], anc[1:2], anc[2:3], anc[3:4]
    al = acx - aw * 0.5
    at = acy - ah * 0.5
    ar = acx + aw * 0.5
    ab = acy + ah * 0.5
    area_a = (ar - al) * (ab - at)

    v_rows = []
    np_rows = []
    sl1_rows = []
    cep_rows = []
    for i in range(_IMGS):
        gt = gt_ref[i]
        gl = gt[:, 0:1]
        gtp = gt[:, 1:2]
        gr = gt[:, 2:3]
        gb = gt[:, 3:4]
        glab = gt[:, 4:5]

        w = jnp.maximum(jnp.minimum(gr, ar) - jnp.maximum(gl, al), 0.0)
        h = jnp.maximum(jnp.minimum(gb, ab) - jnp.maximum(gtp, at), 0.0)
        inter = w * h
        area_g = (gr - gl) * (gb - gtp)
        iou = inter / jnp.maximum(area_g + area_a - inter, 1e-8)
        iou = jnp.where(glab > 0.0, iou, -1.0)

        best = jnp.max(iou, axis=0, keepdims=True)
        mask_pos = best >= _IOU_T

        gidx = jax.lax.broadcasted_iota(jnp.int32, (G, A), 0)
        best_idx = jnp.min(jnp.where(iou == best, gidx, G), axis=0,
                           keepdims=True)
        sel = gidx == best_idx

        selb = jnp.where(sel, 1.0, 0.0).astype(jnp.bfloat16)
        gtt = gtt_ref[i]
        tl, tt, tr, tb, tlab = (gtt[0:1], gtt[1:2], gtt[2:3], gtt[3:4],
                                gtt[4:5])
        coords = jnp.concatenate(
            [(tl + tr) * 0.5, (tt + tb) * 0.5, tr - tl, tb - tt], axis=0)
        hi = coords.astype(jnp.bfloat16)
        r1 = coords - hi.astype(jnp.float32)
        mid = r1.astype(jnp.bfloat16)
        lo = (r1 - mid.astype(jnp.float32)).astype(jnp.bfloat16)
        lhs = jnp.concatenate([hi, tlab.astype(jnp.bfloat16), mid, lo],
                              axis=0)
        m = jnp.dot(lhs, selb, preferred_element_type=jnp.float32)
        gcx = m[0:1] + m[5:6] + m[9:10]
        gcy = m[1:2] + m[6:7] + m[10:11]
        gw = m[2:3] + m[7:8] + m[11:12]
        gh = m[3:4] + m[8:9] + m[12:13]
        glabel = jnp.where(mask_pos, m[4:5], 0.0)

        tx = (gcx - acx) / (aw * _VXY)
        ty = (gcy - acy) / (ah * _VXY)
        tw = jnp.log(jnp.maximum(gw, 1e-6) / aw) / _VWH
        th = jnp.log(jnp.maximum(gh, 1e-6) / ah) / _VWH
        d = preg_ref[i] - jnp.concatenate([tx, ty, tw, th], axis=0)
        ad = jnp.abs(d)
        sl1 = jnp.sum(jnp.where(ad < 1.0, 0.5 * d * d, ad - 0.5), axis=0,
                      keepdims=True)

        pc = pcls_ref[i]
        mx = jnp.max(pc, axis=0, keepdims=True)
        lse = mx + jnp.log(jnp.sum(jnp.exp(pc - mx), axis=0, keepdims=True))
        cidx = jax.lax.broadcasted_iota(jnp.int32, (C, A), 0)
        p_at = jnp.sum(jnp.where(cidx == glabel.astype(jnp.int32), pc, 0.0),
                       axis=0, keepdims=True)
        ce = lse - p_at

        np_rows.append(jnp.sum(jnp.where(mask_pos, 1.0, 0.0), axis=1,
                               keepdims=True))
        sl1_rows.append(jnp.sum(jnp.where(mask_pos, sl1, 0.0), axis=1,
                                keepdims=True))
        cep_rows.append(jnp.sum(jnp.where(mask_pos, ce, 0.0), axis=1,
                                keepdims=True))
        v_rows.append(jnp.maximum(jnp.where(mask_pos, 0.0, ce), 0.0))

    v8 = jnp.concatenate(v_rows, axis=0)
    np8 = jnp.concatenate(np_rows, axis=0)
    sl18 = jnp.concatenate(sl1_rows, axis=0)
    cep8 = jnp.concatenate(cep_rows, axis=0)

    vb8 = pltpu.bitcast(v8, jnp.int32)
    kf = _NEG_RATIO * jnp.maximum(np8, _EPS16)
    kfl = jnp.floor(kf)
    k8 = kfl + jnp.where(kf > kfl, 1.0, 0.0)

    def bit_step(i, p):
        cand = p | jax.lax.shift_left(jnp.int32(1), jnp.int32(30) - i)
        cnt = jnp.sum(jnp.where(vb8 >= cand, 1.0, 0.0), axis=1, keepdims=True)
        return jnp.where(cnt >= k8, cand, p)

    p8 = jax.lax.fori_loop(0, 31, bit_step, jnp.zeros((_IMGS, 1), jnp.int32))

    cnt_gt = jnp.sum(jnp.where(vb8 > p8, 1.0, 0.0), axis=1, keepdims=True)
    sum_gt = jnp.sum(jnp.where(vb8 > p8, v8, 0.0), axis=1, keepdims=True)
    t8 = jnp.max(jnp.where(vb8 <= p8, v8, 0.0), axis=1, keepdims=True)
    neg8 = sum_gt + (k8 - cnt_gt) * t8

    lane = jax.lax.broadcasted_iota(jnp.int32, (_IMGS, 8), 1)
    out_ref[:, 0, :] = (jnp.where(lane == 0, np8, 0.0)
                        + jnp.where(lane == 1, sl18, 0.0)
                        + jnp.where(lane == 2, cep8, 0.0)
                        + jnp.where(lane == 3, neg8, 0.0))


def kernel(preg, pcls, ancs_xywh, gboxes_ltrb, glabels):
    B, _, A = preg.shape
    C = pcls.shape[1]
    G = gboxes_ltrb.shape[1]
    anc_t = ancs_xywh.T
    gt = jnp.concatenate(
        [gboxes_ltrb, glabels[..., None].astype(jnp.float32)], axis=-1)
    gtt = jnp.transpose(gt, (0, 2, 1))

    out = pl.pallas_call(
        _ssd_body,
        grid=(B // _IMGS,),
        in_specs=[
            pl.BlockSpec((4, A), lambda b: (0, 0)),
            pl.BlockSpec((_IMGS, 4, A), lambda b: (b, 0, 0)),
            pl.BlockSpec((_IMGS, C, A), lambda b: (b, 0, 0)),
            pl.BlockSpec((_IMGS, G, 5), lambda b: (b, 0, 0)),
            pl.BlockSpec((_IMGS, 5, G), lambda b: (b, 0, 0)),
        ],
        out_specs=pl.BlockSpec((_IMGS, 1, 8), lambda b: (b, 0, 0)),
        out_shape=jax.ShapeDtypeStruct((B, 1, 8), jnp.float32),
        compiler_params=pltpu.CompilerParams(
            dimension_semantics=("parallel",),
            vmem_limit_bytes=56 * 1024 * 1024,
        ),
    )(anc_t, preg, pcls, gt, gtt)

    r = out[:, 0, :]
    n_pos = r[:, 0]
    l_box = r[:, 1].sum() / jnp.maximum(n_pos.sum(), 1.0)
    nums = jnp.maximum(n_pos, _EPS16)
    return l_box + (r[:, 2] / nums).mean() + (r[:, 3] / nums).mean()
